# Initial kernel scaffold; baseline (speedup 1.0000x reference)
import jax
import jax.numpy as jnp
from jax.experimental import pallas as pl


def kernel(h_fast, h_slow, idx, weight, A_states, W_m, W_ih, W_hh, b_ih, b_hh):
    raise NotImplementedError("write your pallas kernel here")


# speedup vs baseline: 5.5154x; 5.5154x over previous
import functools

import jax
import jax.numpy as jnp
from jax import lax
from jax.experimental import pallas as pl
from jax.experimental.pallas import tpu as pltpu

B = 4096
D = 1024
K = 64
BB = 512


def _agg_kernel(hf_ref, hs_ref, idx_ref, w_ref, wm_ref, agg_ref, ws_ref):
    i = pl.program_id(0)

    @pl.when(i == 0)
    def _init():
        agg_ref[...] = jnp.zeros_like(agg_ref)
        ws_ref[...] = jnp.zeros_like(ws_ref)

    hf = hf_ref[...]
    hs = hs_ref[...]
    dn = (((1,), (1,)), ((), ()))
    m = lax.dot_general(hf, wm_ref[:, :D], dn, preferred_element_type=jnp.float32)
    m += lax.dot_general(hs, wm_ref[:, D:], dn, preferred_element_type=jnp.float32)

    kidx = lax.broadcasted_iota(jnp.int32, (K, BB), 0)
    e0 = idx_ref[0:1, :]
    e1 = idx_ref[1:2, :]
    w0 = w_ref[0:1, :]
    w1 = w_ref[1:2, :]
    s_t = jnp.where(kidx == e0, w0, 0.0) + jnp.where(kidx == e1, w1, 0.0)

    agg_ref[...] += lax.dot_general(
        s_t, m, (((1,), (0,)), ((), ())), preferred_element_type=jnp.float32
    )
    ws_ref[...] += jnp.broadcast_to(
        jnp.sum(s_t, axis=1, keepdims=True), (K, 128)
    )


def _gru_kernel(agg_ref, ws_ref, a_ref, wih_ref, whh_ref, bih_ref, bhh_ref,
                out_ref, am_scr, r_scr, z_scr):
    j = pl.program_id(0)
    dn = (((1,), (1,)), ((), ()))

    @pl.when(j == 0)
    def _mean():
        ws = ws_ref[:, 0:1]
        am_scr[...] = agg_ref[...] / (ws + 1e-9)

    am = am_scr[...]
    a = a_ref[...]
    bih = bih_ref[0]
    bhh = bhh_ref[0]
    gi = lax.dot_general(am, wih_ref[...], dn, preferred_element_type=jnp.float32)
    gh = lax.dot_general(a, whh_ref[...], dn, preferred_element_type=jnp.float32)
    g = gi + gh + bih + bhh

    @pl.when(j == 0)
    def _r():
        r_scr[...] = jax.nn.sigmoid(g)

    @pl.when(j == 1)
    def _z():
        z_scr[...] = jax.nn.sigmoid(g)

    @pl.when(j == 2)
    def _n():
        i_n = gi + bih
        h_n = gh + bhh
        n = jnp.tanh(i_n + r_scr[...] * h_n)
        z = z_scr[...]
        new = (1.0 - z) * n + z * a
        used = ws_ref[:, 0:1] > 0.0
        out_ref[...] = jnp.where(used, new, a)


@jax.jit
def kernel(h_fast, h_slow, idx, weight, A_states, W_m, W_ih, W_hh, b_ih, b_hh):
    idx_t = idx.astype(jnp.int32).T
    w_t = weight.T

    grid = B // BB
    agg, wsum = pl.pallas_call(
        _agg_kernel,
        grid=(grid,),
        in_specs=[
            pl.BlockSpec((BB, D), lambda i: (i, 0)),
            pl.BlockSpec((BB, D), lambda i: (i, 0)),
            pl.BlockSpec((2, BB), lambda i: (0, i)),
            pl.BlockSpec((2, BB), lambda i: (0, i)),
            pl.BlockSpec((D, 2 * D), lambda i: (0, 0)),
        ],
        out_specs=[
            pl.BlockSpec((K, D), lambda i: (0, 0)),
            pl.BlockSpec((K, 128), lambda i: (0, 0)),
        ],
        out_shape=[
            jax.ShapeDtypeStruct((K, D), jnp.float32),
            jax.ShapeDtypeStruct((K, 128), jnp.float32),
        ],
        compiler_params=pltpu.CompilerParams(
            dimension_semantics=("arbitrary",),
        ),
    )(h_fast, h_slow, idx_t, w_t, W_m)

    bih2 = b_ih.reshape(3, 1, D)
    bhh2 = b_hh.reshape(3, 1, D)
    updated = pl.pallas_call(
        _gru_kernel,
        grid=(3,),
        in_specs=[
            pl.BlockSpec((K, D), lambda j: (0, 0)),
            pl.BlockSpec((K, 128), lambda j: (0, 0)),
            pl.BlockSpec((K, D), lambda j: (0, 0)),
            pl.BlockSpec((D, D), lambda j: (j, 0)),
            pl.BlockSpec((D, D), lambda j: (j, 0)),
            pl.BlockSpec((1, 1, D), lambda j: (j, 0, 0)),
            pl.BlockSpec((1, 1, D), lambda j: (j, 0, 0)),
        ],
        out_specs=pl.BlockSpec((K, D), lambda j: (0, 0)),
        out_shape=jax.ShapeDtypeStruct((K, D), jnp.float32),
        scratch_shapes=[
            pltpu.VMEM((K, D), jnp.float32),
            pltpu.VMEM((K, D), jnp.float32),
            pltpu.VMEM((K, D), jnp.float32),
        ],
        compiler_params=pltpu.CompilerParams(
            dimension_semantics=("arbitrary",),
        ),
    )(agg, wsum, A_states, W_ih, W_hh, bih2, bhh2)
    return updated
